# baseline (speedup 1.0000x reference)
import functools

import jax
import jax.numpy as jnp
from jax import lax
from jax.experimental import pallas as pl
from jax.experimental.pallas import tpu as pltpu
from jax.experimental.pallas import tpu_sc as plsc

NC = 2
NS = 16
NW = NC * NS

BATCH = 16384
EMB = 64
B_PER_W = BATCH // NW


def _sc_gather(user_table, item_table, user_ids, item_ids):
    mesh = plsc.VectorSubcoreMesh(core_axis_name="c", subcore_axis_name="s")

    @functools.partial(
        pl.kernel,
        mesh=mesh,
        compiler_params=pltpu.CompilerParams(use_tc_tiling_on_sc=True),
        out_type=[
            jax.ShapeDtypeStruct((BATCH, EMB), jnp.float32),
            jax.ShapeDtypeStruct((BATCH, EMB), jnp.float32),
        ],
        scratch_types=[
            pltpu.VMEM((B_PER_W,), jnp.int32),
            pltpu.VMEM((B_PER_W,), jnp.int32),
            pltpu.SemaphoreType.DMA,
            pltpu.SemaphoreType.DMA,
        ],
    )
    def gather_kernel(utab_hbm, itab_hbm, uid_hbm, iid_hbm,
                      uemb_hbm, iemb_hbm,
                      uids_v, iids_v, idsem, rowsem):
        wid = lax.axis_index("s") * NC + lax.axis_index("c")
        base = wid * B_PER_W
        pltpu.async_copy(uid_hbm.at[pl.ds(base, B_PER_W)], uids_v, idsem).wait()
        pltpu.async_copy(iid_hbm.at[pl.ds(base, B_PER_W)], iids_v, idsem).wait()

        @pl.loop(0, B_PER_W, step=16)
        def _fire(k):
            uv = uids_v[pl.ds(k, 16)]
            iv = iids_v[pl.ds(k, 16)]
            for j in range(16):
                pltpu.async_copy(utab_hbm.at[uv[j]],
                                 uemb_hbm.at[base + k + j], rowsem)
                pltpu.async_copy(itab_hbm.at[iv[j]],
                                 iemb_hbm.at[base + k + j], rowsem)

        @pl.loop(0, 2 * B_PER_W)
        def _drain(k):
            pltpu.make_async_copy(utab_hbm.at[0], uemb_hbm.at[base],
                                  rowsem).wait()

    return gather_kernel(user_table, item_table, user_ids, item_ids)


BM = 1024


def _mlp_body(ue_ref, ie_ref, w1_ref, b1_ref, w2_ref, b2_ref, w3_ref, b3_ref,
              o_ref):
    ue = ue_ref[...]
    ie = ie_ref[...]
    h = jnp.dot(ue, w1_ref[0:EMB, :], preferred_element_type=jnp.float32)
    h = h + jnp.dot(ie, w1_ref[EMB:2 * EMB, :],
                    preferred_element_type=jnp.float32)
    h = jnp.maximum(h + b1_ref[...], 0.0)
    h2 = jnp.dot(h, w2_ref[...], preferred_element_type=jnp.float32)
    h2 = jnp.maximum(h2 + b2_ref[...], 0.0)
    out = jnp.dot(h2, w3_ref[...], preferred_element_type=jnp.float32)
    o_ref[...] = out + b3_ref[...]


def _tc_mlp(uemb, iemb, W1, b1, W2, b2, W3, b3):
    h1 = W1.shape[1]
    h2 = W2.shape[1]
    grid = (BATCH // BM,)
    out = pl.pallas_call(
        _mlp_body,
        grid=grid,
        in_specs=[
            pl.BlockSpec((BM, EMB), lambda i: (i, 0)),
            pl.BlockSpec((BM, EMB), lambda i: (i, 0)),
            pl.BlockSpec((2 * EMB, h1), lambda i: (0, 0)),
            pl.BlockSpec((1, h1), lambda i: (0, 0)),
            pl.BlockSpec((h1, h2), lambda i: (0, 0)),
            pl.BlockSpec((1, h2), lambda i: (0, 0)),
            pl.BlockSpec((h2, 1), lambda i: (0, 0)),
            pl.BlockSpec((1, 1), lambda i: (0, 0)),
        ],
        out_specs=pl.BlockSpec((BM, 1), lambda i: (i, 0)),
        out_shape=jax.ShapeDtypeStruct((BATCH, 1), jnp.float32),
    )(uemb, iemb, W1, b1.reshape(1, h1), W2, b2.reshape(1, h2), W3,
      b3.reshape(1, 1))
    return out.reshape(BATCH)


def kernel(user_ids, item_ids, user_table, item_table, W1, b1, W2, b2, W3, b3):
    uemb, iemb = _sc_gather(user_table, item_table,
                            user_ids.astype(jnp.int32),
                            item_ids.astype(jnp.int32))
    return _tc_mlp(uemb, iemb, W1, b1, W2, b2, W3, b3)


# speedup vs baseline: 1.3090x; 1.3090x over previous
import functools

import jax
import jax.numpy as jnp
from jax import lax
from jax.experimental import pallas as pl
from jax.experimental.pallas import tpu as pltpu
from jax.experimental.pallas import tpu_sc as plsc

NC = 2
NS = 16
NW = NC * NS

BATCH = 16384
EMB = 64
PACK = 2 * EMB
B_PER_W = BATCH // NW


def _sc_gather(utab2, itab2, uidx, iidx):
    mesh = plsc.VectorSubcoreMesh(core_axis_name="c", subcore_axis_name="s")

    @functools.partial(
        pl.kernel,
        mesh=mesh,
        out_type=[
            jax.ShapeDtypeStruct((BATCH, PACK), jnp.float32),
            jax.ShapeDtypeStruct((BATCH, PACK), jnp.float32),
        ],
        scratch_types=[
            pltpu.VMEM((B_PER_W,), jnp.int32),
            pltpu.VMEM((B_PER_W,), jnp.int32),
            pltpu.VMEM((B_PER_W // 2, PACK), jnp.float32),
            pltpu.VMEM((B_PER_W // 2, PACK), jnp.float32),
            pltpu.SemaphoreType.DMA,
            pltpu.SemaphoreType.DMA,
        ],
    )
    def gather_kernel(utab_hbm, itab_hbm, uid_hbm, iid_hbm,
                      uemb_hbm, iemb_hbm,
                      uidx_v, iidx_v, bufa, bufb, sema, semb):
        wid = lax.axis_index("s") * NC + lax.axis_index("c")
        base = wid * B_PER_W
        half = B_PER_W // 2
        pltpu.sync_copy(uid_hbm.at[pl.ds(base, B_PER_W)], uidx_v)
        pltpu.sync_copy(iid_hbm.at[pl.ds(base, B_PER_W)], iidx_v)
        cu0 = pltpu.async_copy(utab_hbm.at[uidx_v.at[pl.ds(0, half)]],
                               bufa, sema)
        cu1 = pltpu.async_copy(utab_hbm.at[uidx_v.at[pl.ds(half, half)]],
                               bufb, semb)
        cu0.wait()
        pltpu.sync_copy(bufa, uemb_hbm.at[pl.ds(base, half)])
        ci0 = pltpu.async_copy(itab_hbm.at[iidx_v.at[pl.ds(0, half)]],
                               bufa, sema)
        cu1.wait()
        pltpu.sync_copy(bufb, uemb_hbm.at[pl.ds(base + half, half)])
        ci1 = pltpu.async_copy(itab_hbm.at[iidx_v.at[pl.ds(half, half)]],
                               bufb, semb)
        ci0.wait()
        pltpu.sync_copy(bufa, iemb_hbm.at[pl.ds(base, half)])
        ci1.wait()
        pltpu.sync_copy(bufb, iemb_hbm.at[pl.ds(base + half, half)])

    return gather_kernel(utab2, itab2, uidx, iidx)


BM = 1024


def _mlp_body(gu_ref, gi_ref, up_ref, ip_ref, w1_ref, b1_ref, w2_ref, b2_ref,
              w3_ref, b3_ref, o_ref):
    gu = gu_ref[...]
    gi = gi_ref[...]
    umask = up_ref[...] != 0
    imask = ip_ref[...] != 0
    ue = jnp.where(umask, gu[:, EMB:PACK], gu[:, 0:EMB])
    ie = jnp.where(imask, gi[:, EMB:PACK], gi[:, 0:EMB])
    h = jnp.dot(ue, w1_ref[0:EMB, :], preferred_element_type=jnp.float32)
    h = h + jnp.dot(ie, w1_ref[EMB:PACK, :],
                    preferred_element_type=jnp.float32)
    h = jnp.maximum(h + b1_ref[...], 0.0)
    h2 = jnp.dot(h, w2_ref[...], preferred_element_type=jnp.float32)
    h2 = jnp.maximum(h2 + b2_ref[...], 0.0)
    out = jnp.dot(h2, w3_ref[...], preferred_element_type=jnp.float32)
    o_ref[...] = out + b3_ref[...]


def _tc_mlp(gu, gi, upar, ipar, W1, b1, W2, b2, W3, b3):
    h1 = W1.shape[1]
    h2 = W2.shape[1]
    grid = (BATCH // BM,)
    out = pl.pallas_call(
        _mlp_body,
        grid=grid,
        in_specs=[
            pl.BlockSpec((BM, PACK), lambda i: (i, 0)),
            pl.BlockSpec((BM, PACK), lambda i: (i, 0)),
            pl.BlockSpec((BM, 1), lambda i: (i, 0)),
            pl.BlockSpec((BM, 1), lambda i: (i, 0)),
            pl.BlockSpec((PACK, h1), lambda i: (0, 0)),
            pl.BlockSpec((1, h1), lambda i: (0, 0)),
            pl.BlockSpec((h1, h2), lambda i: (0, 0)),
            pl.BlockSpec((1, h2), lambda i: (0, 0)),
            pl.BlockSpec((h2, 1), lambda i: (0, 0)),
            pl.BlockSpec((1, 1), lambda i: (0, 0)),
        ],
        out_specs=pl.BlockSpec((BM, 1), lambda i: (i, 0)),
        out_shape=jax.ShapeDtypeStruct((BATCH, 1), jnp.float32),
    )(gu, gi, upar, ipar, W1, b1.reshape(1, h1), W2, b2.reshape(1, h2), W3,
      b3.reshape(1, 1))
    return out.reshape(BATCH)


def kernel(user_ids, item_ids, user_table, item_table, W1, b1, W2, b2, W3, b3):
    uid = user_ids.astype(jnp.int32)
    iid = item_ids.astype(jnp.int32)
    utab2 = user_table.reshape(user_table.shape[0] // 2, PACK)
    itab2 = item_table.reshape(item_table.shape[0] // 2, PACK)
    gu, gi = _sc_gather(utab2, itab2, uid >> 1, iid >> 1)
    return _tc_mlp(gu, gi, (uid & 1).reshape(BATCH, 1),
                   (iid & 1).reshape(BATCH, 1), W1, b1, W2, b2, W3, b3)


# speedup vs baseline: 2.1571x; 1.6479x over previous
import functools

import jax
import jax.numpy as jnp
from jax import lax
from jax.experimental import pallas as pl
from jax.experimental.pallas import tpu as pltpu
from jax.experimental.pallas import tpu_sc as plsc

NC = 2
NS = 16
NW = NC * NS

BATCH = 16384
EMB = 64
PACK = 2 * EMB
B_PER_W = BATCH // NW


def _sc_gather(utab2, itab2, uidx, iidx):
    mesh = plsc.VectorSubcoreMesh(core_axis_name="c", subcore_axis_name="s")

    @functools.partial(
        pl.kernel,
        mesh=mesh,
        out_type=[
            jax.ShapeDtypeStruct((BATCH, PACK), jnp.float32),
            jax.ShapeDtypeStruct((BATCH, PACK), jnp.float32),
        ],
        scratch_types=[
            pltpu.VMEM((B_PER_W,), jnp.int32),
            pltpu.VMEM((B_PER_W,), jnp.int32),
            pltpu.VMEM((B_PER_W // 2, PACK), jnp.float32),
            pltpu.VMEM((B_PER_W // 2, PACK), jnp.float32),
            pltpu.SemaphoreType.DMA,
            pltpu.SemaphoreType.DMA,
        ],
    )
    def gather_kernel(utab_hbm, itab_hbm, uid_hbm, iid_hbm,
                      uemb_hbm, iemb_hbm,
                      uidx_v, iidx_v, bufa, bufb, sema, semb):
        wid = lax.axis_index("s") * NC + lax.axis_index("c")
        base = wid * B_PER_W
        half = B_PER_W // 2
        pltpu.sync_copy(uid_hbm.at[pl.ds(base, B_PER_W)], uidx_v)
        pltpu.sync_copy(iid_hbm.at[pl.ds(base, B_PER_W)], iidx_v)
        cu0 = pltpu.async_copy(utab_hbm.at[uidx_v.at[pl.ds(0, half)]],
                               bufa, sema)
        cu1 = pltpu.async_copy(utab_hbm.at[uidx_v.at[pl.ds(half, half)]],
                               bufb, semb)
        cu0.wait()
        pltpu.sync_copy(bufa, uemb_hbm.at[pl.ds(base, half)])
        ci0 = pltpu.async_copy(itab_hbm.at[iidx_v.at[pl.ds(0, half)]],
                               bufa, sema)
        cu1.wait()
        pltpu.sync_copy(bufb, uemb_hbm.at[pl.ds(base + half, half)])
        ci1 = pltpu.async_copy(itab_hbm.at[iidx_v.at[pl.ds(half, half)]],
                               bufb, semb)
        ci0.wait()
        pltpu.sync_copy(bufa, iemb_hbm.at[pl.ds(base, half)])
        ci1.wait()
        pltpu.sync_copy(bufb, iemb_hbm.at[pl.ds(base + half, half)])

    return gather_kernel(utab2, itab2, uidx, iidx)


PCB = 2048


def _pack_body(lo_ref, hi_ref, out_ref):
    out_ref[:, 0:EMB] = lo_ref[...].T
    out_ref[:, EMB:PACK] = hi_ref[...].T


def _pack(tabT):
    n = tabT.shape[1]
    nblk = pl.cdiv(n // 2, PCB)
    m = nblk * PCB
    last = pl.cdiv(n, PCB) - 1

    return pl.pallas_call(
        _pack_body,
        grid=(nblk,),
        in_specs=[
            pl.BlockSpec((EMB, PCB), lambda i: (0, i)),
            pl.BlockSpec((EMB, PCB),
                         lambda i, nblk=nblk, last=last:
                         (0, jnp.minimum(i + nblk, last))),
        ],
        out_specs=pl.BlockSpec((PCB, PACK), lambda i: (i, 0)),
        out_shape=jax.ShapeDtypeStruct((m, PACK), jnp.float32),
    )(tabT, tabT), m


BM = 1024


def _mlp_body(gu_ref, gi_ref, up_ref, ip_ref, w1_ref, b1_ref, w2_ref, b2_ref,
              w3_ref, b3_ref, o_ref):
    gu = gu_ref[...]
    gi = gi_ref[...]
    umask = up_ref[...] != 0
    imask = ip_ref[...] != 0
    ue = jnp.where(umask, gu[:, EMB:PACK], gu[:, 0:EMB])
    ie = jnp.where(imask, gi[:, EMB:PACK], gi[:, 0:EMB])
    h = jnp.dot(ue, w1_ref[0:EMB, :], preferred_element_type=jnp.float32)
    h = h + jnp.dot(ie, w1_ref[EMB:PACK, :],
                    preferred_element_type=jnp.float32)
    h = jnp.maximum(h + b1_ref[...], 0.0)
    h2 = jnp.dot(h, w2_ref[...], preferred_element_type=jnp.float32)
    h2 = jnp.maximum(h2 + b2_ref[...], 0.0)
    out = jnp.dot(h2, w3_ref[...], preferred_element_type=jnp.float32)
    o_ref[...] = out + b3_ref[...]


def _tc_mlp(gu, gi, upar, ipar, W1, b1, W2, b2, W3, b3):
    h1 = W1.shape[1]
    h2 = W2.shape[1]
    grid = (BATCH // BM,)
    out = pl.pallas_call(
        _mlp_body,
        grid=grid,
        in_specs=[
            pl.BlockSpec((BM, PACK), lambda i: (i, 0)),
            pl.BlockSpec((BM, PACK), lambda i: (i, 0)),
            pl.BlockSpec((BM, 1), lambda i: (i, 0)),
            pl.BlockSpec((BM, 1), lambda i: (i, 0)),
            pl.BlockSpec((PACK, h1), lambda i: (0, 0)),
            pl.BlockSpec((1, h1), lambda i: (0, 0)),
            pl.BlockSpec((h1, h2), lambda i: (0, 0)),
            pl.BlockSpec((1, h2), lambda i: (0, 0)),
            pl.BlockSpec((h2, 1), lambda i: (0, 0)),
            pl.BlockSpec((1, 1), lambda i: (0, 0)),
        ],
        out_specs=pl.BlockSpec((BM, 1), lambda i: (i, 0)),
        out_shape=jax.ShapeDtypeStruct((BATCH, 1), jnp.float32),
    )(gu, gi, upar, ipar, W1, b1.reshape(1, h1), W2, b2.reshape(1, h2), W3,
      b3.reshape(1, 1))
    return out.reshape(BATCH)


def kernel(user_ids, item_ids, user_table, item_table, W1, b1, W2, b2, W3, b3):
    uid = user_ids.astype(jnp.int32)
    iid = item_ids.astype(jnp.int32)
    utab2, mu = _pack(user_table.T)
    itab2, mi = _pack(item_table.T)
    uhigh = uid >= mu
    ihigh = iid >= mi
    uj = jnp.where(uhigh, uid - mu, uid)
    ij = jnp.where(ihigh, iid - mi, iid)
    gu, gi = _sc_gather(utab2, itab2, uj, ij)
    return _tc_mlp(gu, gi, uhigh.astype(jnp.int32).reshape(BATCH, 1),
                   ihigh.astype(jnp.int32).reshape(BATCH, 1),
                   W1, b1, W2, b2, W3, b3)


# speedup vs baseline: 2.8993x; 1.3441x over previous
import functools

import jax
import jax.numpy as jnp
from jax import lax
from jax.experimental import pallas as pl
from jax.experimental.pallas import tpu as pltpu
from jax.experimental.pallas import tpu_sc as plsc

NC = 2
NS = 16
NW = NC * NS

BATCH = 16384
EMB = 64
PACK = 2 * EMB
B_PER_W = BATCH // NW

PCB = 8192


def _pack_body(lo_ref, hi_ref, out_ref):
    out_ref[:, 0:EMB] = lo_ref[...].T
    out_ref[:, EMB:PACK] = hi_ref[...].T


def _pack(tabT):
    n = tabT.shape[1]
    nblk = pl.cdiv(n // 2, PCB)
    m = nblk * PCB
    last = pl.cdiv(n, PCB) - 1

    return pl.pallas_call(
        _pack_body,
        grid=(nblk,),
        in_specs=[
            pl.BlockSpec((EMB, PCB), lambda i: (0, i)),
            pl.BlockSpec((EMB, PCB),
                         lambda i, nblk=nblk, last=last:
                         (0, jnp.minimum(i + nblk, last))),
        ],
        out_specs=pl.BlockSpec((PCB, PACK), lambda i: (i, 0)),
        out_shape=jax.ShapeDtypeStruct((m, PACK), jnp.float32),
    )(tabT, tabT), m


def _sc_gather(tab2, idx):
    mesh = plsc.VectorSubcoreMesh(core_axis_name="c", subcore_axis_name="s")

    @functools.partial(
        pl.kernel,
        mesh=mesh,
        out_type=jax.ShapeDtypeStruct((BATCH, PACK), jnp.float32),
        scratch_types=[
            pltpu.VMEM((B_PER_W,), jnp.int32),
            pltpu.VMEM((B_PER_W, PACK), jnp.float32),
            pltpu.SemaphoreType.DMA,
        ],
    )
    def gather_kernel(tab_hbm, idx_hbm, emb_hbm, idx_v, rows_v, sem):
        wid = lax.axis_index("s") * NC + lax.axis_index("c")
        base = wid * B_PER_W
        pltpu.sync_copy(idx_hbm.at[pl.ds(base, B_PER_W)], idx_v)
        pltpu.async_copy(tab_hbm.at[idx_v], rows_v, sem).wait()
        pltpu.sync_copy(rows_v, emb_hbm.at[pl.ds(base, B_PER_W)])

    return gather_kernel(tab2, idx)


BM = 1024


def _mlp_body(gu_ref, gi_ref, up_ref, ip_ref, w1_ref, b1_ref, w2_ref, b2_ref,
              w3_ref, b3_ref, o_ref):
    gu = gu_ref[...]
    gi = gi_ref[...]
    umask = up_ref[...] != 0
    imask = ip_ref[...] != 0
    ue = jnp.where(umask, gu[:, EMB:PACK], gu[:, 0:EMB])
    ie = jnp.where(imask, gi[:, EMB:PACK], gi[:, 0:EMB])
    h = jnp.dot(ue, w1_ref[0:EMB, :], preferred_element_type=jnp.float32)
    h = h + jnp.dot(ie, w1_ref[EMB:PACK, :],
                    preferred_element_type=jnp.float32)
    h = jnp.maximum(h + b1_ref[...], 0.0)
    h2 = jnp.dot(h, w2_ref[...], preferred_element_type=jnp.float32)
    h2 = jnp.maximum(h2 + b2_ref[...], 0.0)
    out = jnp.dot(h2, w3_ref[...], preferred_element_type=jnp.float32)
    o_ref[...] = out + b3_ref[...]


def _tc_mlp(gu, gi, upar, ipar, W1, b1, W2, b2, W3, b3):
    h1 = W1.shape[1]
    h2 = W2.shape[1]
    grid = (BATCH // BM,)
    out = pl.pallas_call(
        _mlp_body,
        grid=grid,
        in_specs=[
            pl.BlockSpec((BM, PACK), lambda i: (i, 0)),
            pl.BlockSpec((BM, PACK), lambda i: (i, 0)),
            pl.BlockSpec((BM, 1), lambda i: (i, 0)),
            pl.BlockSpec((BM, 1), lambda i: (i, 0)),
            pl.BlockSpec((PACK, h1), lambda i: (0, 0)),
            pl.BlockSpec((1, h1), lambda i: (0, 0)),
            pl.BlockSpec((h1, h2), lambda i: (0, 0)),
            pl.BlockSpec((1, h2), lambda i: (0, 0)),
            pl.BlockSpec((h2, 1), lambda i: (0, 0)),
            pl.BlockSpec((1, 1), lambda i: (0, 0)),
        ],
        out_specs=pl.BlockSpec((BM, 1), lambda i: (i, 0)),
        out_shape=jax.ShapeDtypeStruct((BATCH, 1), jnp.float32),
    )(gu, gi, upar, ipar, W1, b1.reshape(1, h1), W2, b2.reshape(1, h2), W3,
      b3.reshape(1, 1))
    return out.reshape(BATCH)


def kernel(user_ids, item_ids, user_table, item_table, W1, b1, W2, b2, W3, b3):
    uid = user_ids.astype(jnp.int32)
    iid = item_ids.astype(jnp.int32)
    itab2, mi = _pack(item_table.T)
    ihigh = iid >= mi
    gi = _sc_gather(itab2, jnp.where(ihigh, iid - mi, iid))
    utab2, mu = _pack(user_table.T)
    uhigh = uid >= mu
    gu = _sc_gather(utab2, jnp.where(uhigh, uid - mu, uid))
    return _tc_mlp(gu, gi, uhigh.astype(jnp.int32).reshape(BATCH, 1),
                   ihigh.astype(jnp.int32).reshape(BATCH, 1),
                   W1, b1, W2, b2, W3, b3)
